# Initial kernel scaffold; baseline (speedup 1.0000x reference)
import jax
import jax.numpy as jnp
from jax.experimental import pallas as pl


def kernel(x, gate_emb, gate_bias, shared_w1, shared_w2, shared_w3, exp_w1, exp_w2, exp_w3):
    raise NotImplementedError("write your pallas kernel here")


# speedup vs baseline: 5.9720x; 5.9720x over previous
import functools

import jax
import jax.numpy as jnp
from jax import lax
from jax.experimental import pallas as pl
from jax.experimental.pallas import tpu as pltpu
from jax.experimental.pallas import tpu_sc as plsc

SEQ = 2048
DIM = 1024
NEXP = 64
HID = 256
NSHARED = 2
ROWS = 2 * SEQ
TILE = 128
NTILES = ROWS // TILE
GRID_G = NTILES + NEXP - 1
NCORES = 2
NSUB = 16
NW = NCORES * NSUB
TOKW = SEQ // NW
SHTILE = 256


def _route_body(x_ref, ge_ref, gb_ref, w0x_ref, w1x_ref, dst0_ref, dst1_ref,
                cnt_ref, off_ref, ent_ref):
    xf = x_ref[...]
    ge = ge_ref[...]
    logits = lax.dot_general(xf, ge, (((1,), (1,)), ((), ())),
                             preferred_element_type=jnp.float32)
    scores = jax.nn.sigmoid(logits) + gb_ref[...]

    eidx = lax.broadcasted_iota(jnp.int32, (SEQ, NEXP), 1)
    m1 = jnp.max(scores, axis=1, keepdims=True)
    i1 = jnp.min(jnp.where(scores == m1, eidx, NEXP), axis=1, keepdims=True)
    scores2 = jnp.where(eidx == i1, -jnp.inf, scores)
    m2 = jnp.max(scores2, axis=1, keepdims=True)
    i2 = jnp.min(jnp.where(scores2 == m2, eidx, NEXP), axis=1, keepdims=True)

    denom = m1 + m2
    w0 = m1 / denom
    w1 = m2 / denom
    ent = -(w0 * jnp.log(w0) + w1 * jnp.log(w1))
    ent_ref[...] = jnp.broadcast_to(jnp.mean(ent), (1, 1))

    oh0 = (eidx == i1).astype(jnp.float32)
    oh1 = (eidx == i2).astype(jnp.float32)
    comb = oh0 + oh1

    ri = lax.broadcasted_iota(jnp.int32, (SEQ, SEQ), 0)
    ci = lax.broadcasted_iota(jnp.int32, (SEQ, SEQ), 1)
    tri = (ri > ci).astype(jnp.float32)
    cex = lax.dot_general(tri, comb, (((1,), (0,)), ((), ())),
                          preferred_element_type=jnp.float32)
    counts = cex[SEQ - 1:SEQ, :] + comb[SEQ - 1:SEQ, :]

    inc = counts
    for sh in (1, 2, 4, 8, 16, 32):
        shifted = jnp.concatenate(
            [jnp.zeros((1, sh), jnp.float32), inc[:, :NEXP - sh]], axis=1)
        inc = inc + shifted
    off = inc - counts

    pos = off + cex
    dst0 = jnp.sum(pos * oh0, axis=1, keepdims=True)
    dst1 = jnp.sum(pos * oh1, axis=1, keepdims=True)

    cnt_ref[...] = jnp.round(counts).astype(jnp.int32)
    off_ref[...] = jnp.round(off).astype(jnp.int32)
    dst0_ref[...] = jnp.round(dst0).astype(jnp.int32)
    dst1_ref[...] = jnp.round(dst1).astype(jnp.int32)
    w0x_ref[...] = jnp.broadcast_to(w0, (SEQ, 16))
    w1x_ref[...] = jnp.broadcast_to(w1, (SEQ, 16))


def _route_call(xf, gate_emb, gate_bias2d):
    out_shape = (
        jax.ShapeDtypeStruct((SEQ, 16), jnp.float32),
        jax.ShapeDtypeStruct((SEQ, 16), jnp.float32),
        jax.ShapeDtypeStruct((SEQ, 1), jnp.int32),
        jax.ShapeDtypeStruct((SEQ, 1), jnp.int32),
        jax.ShapeDtypeStruct((1, NEXP), jnp.int32),
        jax.ShapeDtypeStruct((1, NEXP), jnp.int32),
        jax.ShapeDtypeStruct((1, 1), jnp.float32),
    )
    return pl.pallas_call(_route_body, out_shape=out_shape)(
        xf, gate_emb, gate_bias2d)


def _work_items(counts, starts):
    ends = starts + counts
    tt = jnp.where(counts > 0, (ends - 1) // TILE - starts // TILE + 1, 0)
    item_start = jnp.concatenate(
        [jnp.zeros((1,), jnp.int32), jnp.cumsum(tt).astype(jnp.int32)])
    total = item_start[NEXP]
    g = jnp.arange(GRID_G, dtype=jnp.int32)
    eid = jnp.searchsorted(item_start[1:], g, side='right').astype(jnp.int32)
    eid = jnp.minimum(eid, NEXP - 1)
    tid = starts[eid] // TILE + (g - item_start[eid])
    valid = (g < total).astype(jnp.int32)
    tid = jnp.where(valid > 0, tid, NTILES - 1).astype(jnp.int32)
    first = jnp.concatenate(
        [jnp.ones((1,), jnp.int32), (tid[1:] != tid[:-1]).astype(jnp.int32)])
    return tid, eid, first, valid


def _ffn_body(tid_ref, eid_ref, first_ref, valid_ref, st_ref, en_ref,
              xs_ref, w1_ref, w3_ref, w2_ref, out_ref):
    g = pl.program_id(0)
    e = eid_ref[g]
    row0 = tid_ref[g] * TILE
    rows = row0 + lax.broadcasted_iota(jnp.int32, (TILE, 1), 0)
    mask = ((rows >= st_ref[e]) & (rows < en_ref[e])
            & (valid_ref[g] > 0)).astype(jnp.float32)
    x = xs_ref[...] * mask
    h1 = lax.dot_general(x, w1_ref[0], (((1,), (1,)), ((), ())),
                         preferred_element_type=jnp.float32)
    h3 = lax.dot_general(x, w3_ref[0], (((1,), (1,)), ((), ())),
                         preferred_element_type=jnp.float32)
    hh = h1 * jax.nn.sigmoid(h1) * h3
    contrib = lax.dot_general(hh, w2_ref[0], (((1,), (1,)), ((), ())),
                              preferred_element_type=jnp.float32)

    @pl.when(first_ref[g] > 0)
    def _():
        out_ref[...] = contrib

    @pl.when(first_ref[g] == 0)
    def _():
        out_ref[...] += contrib


def _ffn_call(tid, eid, first, valid, starts, ends, xs, exp_w1, exp_w3, exp_w2):
    grid_spec = pltpu.PrefetchScalarGridSpec(
        num_scalar_prefetch=6,
        grid=(GRID_G,),
        in_specs=[
            pl.BlockSpec((TILE, DIM), lambda g, tid, eid, f, v, st, en: (tid[g], 0)),
            pl.BlockSpec((1, HID, DIM), lambda g, tid, eid, f, v, st, en: (eid[g], 0, 0)),
            pl.BlockSpec((1, HID, DIM), lambda g, tid, eid, f, v, st, en: (eid[g], 0, 0)),
            pl.BlockSpec((1, DIM, HID), lambda g, tid, eid, f, v, st, en: (eid[g], 0, 0)),
        ],
        out_specs=pl.BlockSpec((TILE, DIM), lambda g, tid, eid, f, v, st, en: (tid[g], 0)),
    )
    return pl.pallas_call(
        _ffn_body,
        grid_spec=grid_spec,
        out_shape=jax.ShapeDtypeStruct((ROWS, DIM), jnp.float32),
        compiler_params=pltpu.CompilerParams(
            dimension_semantics=("arbitrary",)),
    )(tid, eid, first, valid, starts, ends, xs, exp_w1, exp_w3, exp_w2)


def _shared_body(x_ref, w1_ref, w2_ref, w3_ref, out_ref):
    x = x_ref[...]
    acc = None
    for s in range(NSHARED):
        h1 = lax.dot_general(x, w1_ref[s], (((1,), (1,)), ((), ())),
                             preferred_element_type=jnp.float32)
        h3 = lax.dot_general(x, w3_ref[s], (((1,), (1,)), ((), ())),
                             preferred_element_type=jnp.float32)
        hh = h1 * jax.nn.sigmoid(h1) * h3
        o = lax.dot_general(hh, w2_ref[s], (((1,), (1,)), ((), ())),
                            preferred_element_type=jnp.float32)
        acc = o if acc is None else acc + o
    out_ref[...] = acc * (1.0 / NSHARED)


def _shared_call(xf, shared_w1, shared_w2, shared_w3):
    nst = SEQ // SHTILE
    return pl.pallas_call(
        _shared_body,
        grid=(nst,),
        in_specs=[
            pl.BlockSpec((SHTILE, DIM), lambda i: (i, 0)),
            pl.BlockSpec((NSHARED, HID, DIM), lambda i: (0, 0, 0)),
            pl.BlockSpec((NSHARED, DIM, HID), lambda i: (0, 0, 0)),
            pl.BlockSpec((NSHARED, HID, DIM), lambda i: (0, 0, 0)),
        ],
        out_specs=pl.BlockSpec((SHTILE, DIM), lambda i: (i, 0)),
        out_shape=jax.ShapeDtypeStruct((SEQ, DIM), jnp.float32),
    )(xf, shared_w1, shared_w2, shared_w3)


def _dispatch_body(xf_hbm, dst0_hbm, dst1_hbm, xs_hbm, idx_v, rows_v, sem):
    wid = lax.axis_index("s") * NCORES + lax.axis_index("c")
    base = wid * TOKW
    pltpu.sync_copy(xf_hbm.at[pl.ds(base, TOKW)], rows_v)
    pltpu.sync_copy(dst0_hbm.at[wid], idx_v)
    pltpu.async_copy(rows_v, xs_hbm.at[idx_v], sem).wait()
    pltpu.sync_copy(dst1_hbm.at[wid], idx_v)
    pltpu.async_copy(rows_v, xs_hbm.at[idx_v], sem).wait()


def _dispatch_call(xf, dst0w, dst1w):
    mesh = plsc.VectorSubcoreMesh(core_axis_name="c", subcore_axis_name="s")
    f = pl.kernel(
        _dispatch_body,
        out_type=jax.ShapeDtypeStruct((ROWS, DIM), jnp.float32),
        mesh=mesh,
        scratch_types=[
            pltpu.VMEM((TOKW,), jnp.int32),
            pltpu.VMEM((TOKW, DIM), jnp.float32),
            pltpu.SemaphoreType.DMA,
        ],
    )
    return f(xf, dst0w, dst1w)


def _combine_body(ys_hbm, sh_hbm, dst0_hbm, dst1_hbm, w0x_hbm, w1x_hbm,
                  out_hbm, idx0_v, idx1_v, w0_v, w1_v, acc_v, tmp_v, sem):
    wid = lax.axis_index("s") * NCORES + lax.axis_index("c")
    base = wid * TOKW
    half = TOKW // 2
    pltpu.sync_copy(dst0_hbm.at[wid], idx0_v)
    pltpu.sync_copy(dst1_hbm.at[wid], idx1_v)
    pltpu.sync_copy(w0x_hbm.at[pl.ds(base, TOKW)], w0_v)
    pltpu.sync_copy(w1x_hbm.at[pl.ds(base, TOKW)], w1_v)

    for h in range(2):
        hb = h * half
        pltpu.sync_copy(sh_hbm.at[pl.ds(base + hb, half)], acc_v)

        for slot in range(2):
            idx_v = idx0_v if slot == 0 else idx1_v
            wv = w0_v if slot == 0 else w1_v
            pltpu.async_copy(ys_hbm.at[idx_v.at[h]], tmp_v, sem).wait()

            def row_body(r, _, wv=wv, hb=hb):
                wb = wv[hb + r]
                for c in range(DIM // 16):
                    sl = pl.ds(c * 16, 16)
                    acc_v[r, sl] = acc_v[r, sl] + wb * tmp_v[r, sl]
                return 0

            lax.fori_loop(0, half, row_body, 0)

        pltpu.sync_copy(acc_v, out_hbm.at[pl.ds(base + hb, half)])


def _combine_call(ys, shared, dst0h, dst1h, w0x, w1x):
    mesh = plsc.VectorSubcoreMesh(core_axis_name="c", subcore_axis_name="s")
    half = TOKW // 2
    f = pl.kernel(
        _combine_body,
        out_type=jax.ShapeDtypeStruct((SEQ, DIM), jnp.float32),
        mesh=mesh,
        scratch_types=[
            pltpu.VMEM((2, half), jnp.int32),
            pltpu.VMEM((2, half), jnp.int32),
            pltpu.VMEM((TOKW, 16), jnp.float32),
            pltpu.VMEM((TOKW, 16), jnp.float32),
            pltpu.VMEM((half, DIM), jnp.float32),
            pltpu.VMEM((half, DIM), jnp.float32),
            pltpu.SemaphoreType.DMA,
        ],
    )
    return f(ys, shared, dst0h, dst1h, w0x, w1x)


def kernel(x, gate_emb, gate_bias, shared_w1, shared_w2, shared_w3,
           exp_w1, exp_w2, exp_w3):
    b, s, d = x.shape
    xf = x.reshape(SEQ, DIM)

    w0x, w1x, dst0, dst1, cnt, off, ent = _route_call(
        xf, gate_emb, gate_bias.reshape(1, NEXP))

    shared = _shared_call(xf, shared_w1, shared_w2, shared_w3)

    dst0f = dst0[:, 0]
    dst1f = dst1[:, 0]
    xs = _dispatch_call(xf, dst0f.reshape(NW, TOKW), dst1f.reshape(NW, TOKW))

    counts = cnt[0]
    starts = off[0]
    tid, eid, first, valid = _work_items(counts, starts)
    ends = starts + counts
    ys = _ffn_call(tid, eid, first, valid, starts, ends,
                   xs, exp_w1, exp_w3, exp_w2)

    half = TOKW // 2
    out = _combine_call(ys, shared,
                        dst0f.reshape(NW, 2, half), dst1f.reshape(NW, 2, half),
                        w0x, w1x)

    aux_loss = jnp.asarray(0.0, dtype=x.dtype)
    return out.reshape(b, s, d), aux_loss, ent[0, 0]


# speedup vs baseline: 6.6761x; 1.1179x over previous
import functools

import jax
import jax.numpy as jnp
from jax import lax
from jax.experimental import pallas as pl
from jax.experimental.pallas import tpu as pltpu
from jax.experimental.pallas import tpu_sc as plsc

SEQ = 2048
DIM = 1024
NEXP = 64
HID = 256
NSHARED = 2
ROWS = 2 * SEQ
TILE = 128
NTILES = ROWS // TILE
GRID_G = NTILES + NEXP - 1
NCORES = 2
NSUB = 16
NW = NCORES * NSUB
TOKW = SEQ // NW
SHTILE = 256


def _route_body(x_ref, ge_ref, gb_ref, w0x_ref, w1x_ref, dst0_ref, dst1_ref,
                cnt_ref, off_ref, ent_ref):
    xf = x_ref[...]
    ge = ge_ref[...]
    logits = lax.dot_general(xf, ge, (((1,), (1,)), ((), ())),
                             preferred_element_type=jnp.float32)
    scores = jax.nn.sigmoid(logits) + gb_ref[...]

    eidx = lax.broadcasted_iota(jnp.int32, (SEQ, NEXP), 1)
    m1 = jnp.max(scores, axis=1, keepdims=True)
    i1 = jnp.min(jnp.where(scores == m1, eidx, NEXP), axis=1, keepdims=True)
    scores2 = jnp.where(eidx == i1, -jnp.inf, scores)
    m2 = jnp.max(scores2, axis=1, keepdims=True)
    i2 = jnp.min(jnp.where(scores2 == m2, eidx, NEXP), axis=1, keepdims=True)

    denom = m1 + m2
    w0 = m1 / denom
    w1 = m2 / denom
    ent = -(w0 * jnp.log(w0) + w1 * jnp.log(w1))
    ent_ref[...] = jnp.broadcast_to(jnp.mean(ent), (1, 1))

    oh0 = (eidx == i1).astype(jnp.float32)
    oh1 = (eidx == i2).astype(jnp.float32)
    comb = oh0 + oh1

    ri = lax.broadcasted_iota(jnp.int32, (SEQ, SEQ), 0)
    ci = lax.broadcasted_iota(jnp.int32, (SEQ, SEQ), 1)
    tri = (ri > ci).astype(jnp.bfloat16)
    cex = lax.dot_general(tri, comb.astype(jnp.bfloat16),
                          (((1,), (0,)), ((), ())),
                          preferred_element_type=jnp.float32)
    counts = cex[SEQ - 1:SEQ, :] + comb[SEQ - 1:SEQ, :]

    inc = counts
    for sh in (1, 2, 4, 8, 16, 32):
        shifted = jnp.concatenate(
            [jnp.zeros((1, sh), jnp.float32), inc[:, :NEXP - sh]], axis=1)
        inc = inc + shifted
    off = inc - counts

    pos = off + cex
    dst0 = jnp.sum(pos * oh0, axis=1, keepdims=True)
    dst1 = jnp.sum(pos * oh1, axis=1, keepdims=True)

    cnt_ref[...] = jnp.round(counts).astype(jnp.int32)
    off_ref[...] = jnp.round(off).astype(jnp.int32)
    dst0_ref[...] = jnp.round(dst0).astype(jnp.int32)
    dst1_ref[...] = jnp.round(dst1).astype(jnp.int32)
    w0x_ref[...] = jnp.broadcast_to(w0, (SEQ, 16))
    w1x_ref[...] = jnp.broadcast_to(w1, (SEQ, 16))


def _route_call(xf, gate_emb, gate_bias2d):
    out_shape = (
        jax.ShapeDtypeStruct((SEQ, 16), jnp.float32),
        jax.ShapeDtypeStruct((SEQ, 16), jnp.float32),
        jax.ShapeDtypeStruct((SEQ, 1), jnp.int32),
        jax.ShapeDtypeStruct((SEQ, 1), jnp.int32),
        jax.ShapeDtypeStruct((1, NEXP), jnp.int32),
        jax.ShapeDtypeStruct((1, NEXP), jnp.int32),
        jax.ShapeDtypeStruct((1, 1), jnp.float32),
    )
    return pl.pallas_call(_route_body, out_shape=out_shape)(
        xf, gate_emb, gate_bias2d)


def _work_items(counts, starts):
    ends = starts + counts
    tt = jnp.where(counts > 0, (ends - 1) // TILE - starts // TILE + 1, 0)
    item_start = jnp.concatenate(
        [jnp.zeros((1,), jnp.int32), jnp.cumsum(tt).astype(jnp.int32)])
    total = item_start[NEXP]
    g = jnp.arange(GRID_G, dtype=jnp.int32)
    eid = jnp.searchsorted(item_start[1:], g, side='right').astype(jnp.int32)
    eid = jnp.minimum(eid, NEXP - 1)
    tid = starts[eid] // TILE + (g - item_start[eid])
    valid = (g < total).astype(jnp.int32)
    last = jnp.maximum(total - 1, 0)
    tid = jnp.where(valid > 0, tid, tid[last]).astype(jnp.int32)
    eid = jnp.where(valid > 0, eid, eid[last]).astype(jnp.int32)
    first = jnp.concatenate(
        [jnp.ones((1,), jnp.int32), (tid[1:] != tid[:-1]).astype(jnp.int32)])
    first = first * valid
    return tid, eid, first, valid


def _ffn_body(tid_ref, eid_ref, first_ref, valid_ref, st_ref, en_ref,
              xs_ref, w1_ref, w3_ref, w2_ref, out_ref):
    g = pl.program_id(0)

    @pl.when(valid_ref[g] > 0)
    def _():
        e = eid_ref[g]
        row0 = tid_ref[g] * TILE
        rows = row0 + lax.broadcasted_iota(jnp.int32, (TILE, 1), 0)
        mask = ((rows >= st_ref[e]) & (rows < en_ref[e])).astype(jnp.float32)
        x = xs_ref[...] * mask
        h1 = lax.dot_general(x, w1_ref[0], (((1,), (1,)), ((), ())),
                             preferred_element_type=jnp.float32)
        h3 = lax.dot_general(x, w3_ref[0], (((1,), (1,)), ((), ())),
                             preferred_element_type=jnp.float32)
        hh = h1 * jax.nn.sigmoid(h1) * h3
        contrib = lax.dot_general(hh, w2_ref[0], (((1,), (1,)), ((), ())),
                                  preferred_element_type=jnp.float32)

        @pl.when(first_ref[g] > 0)
        def _():
            out_ref[...] = contrib

        @pl.when(first_ref[g] == 0)
        def _():
            out_ref[...] += contrib


def _ffn_call(tid, eid, first, valid, starts, ends, xs, exp_w1, exp_w3, exp_w2):
    grid_spec = pltpu.PrefetchScalarGridSpec(
        num_scalar_prefetch=6,
        grid=(GRID_G,),
        in_specs=[
            pl.BlockSpec((TILE, DIM), lambda g, tid, eid, f, v, st, en: (tid[g], 0)),
            pl.BlockSpec((1, HID, DIM), lambda g, tid, eid, f, v, st, en: (eid[g], 0, 0)),
            pl.BlockSpec((1, HID, DIM), lambda g, tid, eid, f, v, st, en: (eid[g], 0, 0)),
            pl.BlockSpec((1, DIM, HID), lambda g, tid, eid, f, v, st, en: (eid[g], 0, 0)),
        ],
        out_specs=pl.BlockSpec((TILE, DIM), lambda g, tid, eid, f, v, st, en: (tid[g], 0)),
    )
    return pl.pallas_call(
        _ffn_body,
        grid_spec=grid_spec,
        out_shape=jax.ShapeDtypeStruct((ROWS, DIM), jnp.float32),
        compiler_params=pltpu.CompilerParams(
            dimension_semantics=("arbitrary",)),
    )(tid, eid, first, valid, starts, ends, xs, exp_w1, exp_w3, exp_w2)


def _shared_body(x_ref, w1_ref, w2_ref, w3_ref, out_ref):
    x = x_ref[...]
    acc = None
    for s in range(NSHARED):
        h1 = lax.dot_general(x, w1_ref[s], (((1,), (1,)), ((), ())),
                             preferred_element_type=jnp.float32)
        h3 = lax.dot_general(x, w3_ref[s], (((1,), (1,)), ((), ())),
                             preferred_element_type=jnp.float32)
        hh = h1 * jax.nn.sigmoid(h1) * h3
        o = lax.dot_general(hh, w2_ref[s], (((1,), (1,)), ((), ())),
                            preferred_element_type=jnp.float32)
        acc = o if acc is None else acc + o
    out_ref[...] = acc * (1.0 / NSHARED)


def _shared_call(xf, shared_w1, shared_w2, shared_w3):
    nst = SEQ // SHTILE
    return pl.pallas_call(
        _shared_body,
        grid=(nst,),
        in_specs=[
            pl.BlockSpec((SHTILE, DIM), lambda i: (i, 0)),
            pl.BlockSpec((NSHARED, HID, DIM), lambda i: (0, 0, 0)),
            pl.BlockSpec((NSHARED, DIM, HID), lambda i: (0, 0, 0)),
            pl.BlockSpec((NSHARED, HID, DIM), lambda i: (0, 0, 0)),
        ],
        out_specs=pl.BlockSpec((SHTILE, DIM), lambda i: (i, 0)),
        out_shape=jax.ShapeDtypeStruct((SEQ, DIM), jnp.float32),
    )(xf, shared_w1, shared_w2, shared_w3)


def _dispatch_body(xf_hbm, dst0_hbm, dst1_hbm, xs_hbm, idx_v, rows_v, sem):
    wid = lax.axis_index("s") * NCORES + lax.axis_index("c")
    base = wid * TOKW
    pltpu.sync_copy(xf_hbm.at[pl.ds(base, TOKW)], rows_v)
    pltpu.sync_copy(dst0_hbm.at[wid], idx_v)
    pltpu.async_copy(rows_v, xs_hbm.at[idx_v], sem).wait()
    pltpu.sync_copy(dst1_hbm.at[wid], idx_v)
    pltpu.async_copy(rows_v, xs_hbm.at[idx_v], sem).wait()


def _dispatch_call(xf, dst0w, dst1w):
    mesh = plsc.VectorSubcoreMesh(core_axis_name="c", subcore_axis_name="s")
    f = pl.kernel(
        _dispatch_body,
        out_type=jax.ShapeDtypeStruct((ROWS, DIM), jnp.float32),
        mesh=mesh,
        scratch_types=[
            pltpu.VMEM((TOKW,), jnp.int32),
            pltpu.VMEM((TOKW, DIM), jnp.float32),
            pltpu.SemaphoreType.DMA,
        ],
    )
    return f(xf, dst0w, dst1w)


def _combine_body(ys_hbm, sh_hbm, dst0_hbm, dst1_hbm, w0x_hbm, w1x_hbm,
                  out_hbm, idx0_v, idx1_v, w0_v, w1_v, acc_v, tmp_v, sem):
    wid = lax.axis_index("s") * NCORES + lax.axis_index("c")
    base = wid * TOKW
    half = TOKW // 2
    pltpu.sync_copy(dst0_hbm.at[wid], idx0_v)
    pltpu.sync_copy(dst1_hbm.at[wid], idx1_v)
    pltpu.sync_copy(w0x_hbm.at[pl.ds(base, TOKW)], w0_v)
    pltpu.sync_copy(w1x_hbm.at[pl.ds(base, TOKW)], w1_v)

    for h in range(2):
        hb = h * half
        pltpu.sync_copy(sh_hbm.at[pl.ds(base + hb, half)], acc_v)

        for slot in range(2):
            idx_v = idx0_v if slot == 0 else idx1_v
            wv = w0_v if slot == 0 else w1_v
            pltpu.async_copy(ys_hbm.at[idx_v.at[h]], tmp_v, sem).wait()

            def row_body(r, _, wv=wv, hb=hb):
                wb = wv[hb + r]
                for c in range(DIM // 16):
                    sl = pl.ds(c * 16, 16)
                    acc_v[r, sl] = acc_v[r, sl] + wb * tmp_v[r, sl]
                return 0

            lax.fori_loop(0, half, row_body, 0)

        pltpu.sync_copy(acc_v, out_hbm.at[pl.ds(base + hb, half)])


def _combine_call(ys, shared, dst0h, dst1h, w0x, w1x):
    mesh = plsc.VectorSubcoreMesh(core_axis_name="c", subcore_axis_name="s")
    half = TOKW // 2
    f = pl.kernel(
        _combine_body,
        out_type=jax.ShapeDtypeStruct((SEQ, DIM), jnp.float32),
        mesh=mesh,
        scratch_types=[
            pltpu.VMEM((2, half), jnp.int32),
            pltpu.VMEM((2, half), jnp.int32),
            pltpu.VMEM((TOKW, 16), jnp.float32),
            pltpu.VMEM((TOKW, 16), jnp.float32),
            pltpu.VMEM((half, DIM), jnp.float32),
            pltpu.VMEM((half, DIM), jnp.float32),
            pltpu.SemaphoreType.DMA,
        ],
    )
    return f(ys, shared, dst0h, dst1h, w0x, w1x)


def kernel(x, gate_emb, gate_bias, shared_w1, shared_w2, shared_w3,
           exp_w1, exp_w2, exp_w3):
    b, s, d = x.shape
    xf = x.reshape(SEQ, DIM)

    w0x, w1x, dst0, dst1, cnt, off, ent = _route_call(
        xf, gate_emb, gate_bias.reshape(1, NEXP))

    shared = _shared_call(xf, shared_w1, shared_w2, shared_w3)

    dst0f = dst0[:, 0]
    dst1f = dst1[:, 0]
    xs = _dispatch_call(xf, dst0f.reshape(NW, TOKW), dst1f.reshape(NW, TOKW))

    counts = cnt[0]
    starts = off[0]
    tid, eid, first, valid = _work_items(counts, starts)
    ends = starts + counts
    ys = _ffn_call(tid, eid, first, valid, starts, ends,
                   xs, exp_w1, exp_w3, exp_w2)

    half = TOKW // 2
    out = _combine_call(ys, shared,
                        dst0f.reshape(NW, 2, half), dst1f.reshape(NW, 2, half),
                        w0x, w1x)

    aux_loss = jnp.asarray(0.0, dtype=x.dtype)
    return out.reshape(b, s, d), aux_loss, ent[0, 0]


# speedup vs baseline: 6.6903x; 1.0021x over previous
import functools

import jax
import jax.numpy as jnp
from jax import lax
from jax.experimental import pallas as pl
from jax.experimental.pallas import tpu as pltpu
from jax.experimental.pallas import tpu_sc as plsc

SEQ = 2048
DIM = 1024
NEXP = 64
HID = 256
NSHARED = 2
ROWS = 2 * SEQ
TILE = 128
NTILES = ROWS // TILE
GRID_G = NTILES + NEXP - 1
NCORES = 2
NSUB = 16
NW = NCORES * NSUB
TOKW = SEQ // NW
SHTILE = 256


def _route_body(x_ref, ge_ref, gb_ref, w0x_ref, w1x_ref, dst0_ref, dst1_ref,
                cnt_ref, off_ref, ent_ref):
    xf = x_ref[...]
    ge = ge_ref[...]
    logits = lax.dot_general(xf, ge, (((1,), (1,)), ((), ())),
                             preferred_element_type=jnp.float32)
    scores = jax.nn.sigmoid(logits) + gb_ref[...]

    eidx = lax.broadcasted_iota(jnp.int32, (SEQ, NEXP), 1)
    m1 = jnp.max(scores, axis=1, keepdims=True)
    i1 = jnp.min(jnp.where(scores == m1, eidx, NEXP), axis=1, keepdims=True)
    scores2 = jnp.where(eidx == i1, -jnp.inf, scores)
    m2 = jnp.max(scores2, axis=1, keepdims=True)
    i2 = jnp.min(jnp.where(scores2 == m2, eidx, NEXP), axis=1, keepdims=True)

    denom = m1 + m2
    w0 = m1 / denom
    w1 = m2 / denom
    ent = -(w0 * jnp.log(w0) + w1 * jnp.log(w1))
    ent_ref[...] = jnp.broadcast_to(jnp.mean(ent), (1, 1))

    oh0 = (eidx == i1).astype(jnp.float32)
    oh1 = (eidx == i2).astype(jnp.float32)
    comb = oh0 + oh1

    ri = lax.broadcasted_iota(jnp.int32, (SEQ, SEQ), 0)
    ci = lax.broadcasted_iota(jnp.int32, (SEQ, SEQ), 1)
    tri = (ri > ci).astype(jnp.bfloat16)
    cex = lax.dot_general(tri, comb.astype(jnp.bfloat16),
                          (((1,), (0,)), ((), ())),
                          preferred_element_type=jnp.float32)
    counts = cex[SEQ - 1:SEQ, :] + comb[SEQ - 1:SEQ, :]

    inc = counts
    for sh in (1, 2, 4, 8, 16, 32):
        shifted = jnp.concatenate(
            [jnp.zeros((1, sh), jnp.float32), inc[:, :NEXP - sh]], axis=1)
        inc = inc + shifted
    off = inc - counts

    pos = off + cex
    dst0 = jnp.sum(pos * oh0, axis=1, keepdims=True)
    dst1 = jnp.sum(pos * oh1, axis=1, keepdims=True)

    cnt_ref[...] = jnp.round(counts).astype(jnp.int32)
    off_ref[...] = jnp.round(off).astype(jnp.int32)
    dst0_ref[...] = jnp.round(dst0).astype(jnp.int32)
    dst1_ref[...] = jnp.round(dst1).astype(jnp.int32)
    w0x_ref[...] = jnp.broadcast_to(w0, (SEQ, 16))
    w1x_ref[...] = jnp.broadcast_to(w1, (SEQ, 16))


def _route_call(xf, gate_emb, gate_bias2d):
    out_shape = (
        jax.ShapeDtypeStruct((SEQ, 16), jnp.float32),
        jax.ShapeDtypeStruct((SEQ, 16), jnp.float32),
        jax.ShapeDtypeStruct((SEQ, 1), jnp.int32),
        jax.ShapeDtypeStruct((SEQ, 1), jnp.int32),
        jax.ShapeDtypeStruct((1, NEXP), jnp.int32),
        jax.ShapeDtypeStruct((1, NEXP), jnp.int32),
        jax.ShapeDtypeStruct((1, 1), jnp.float32),
    )
    return pl.pallas_call(_route_body, out_shape=out_shape)(
        xf, gate_emb, gate_bias2d)


def _work_items(counts, starts):
    ends = starts + counts
    tt = jnp.where(counts > 0, (ends - 1) // TILE - starts // TILE + 1, 0)
    item_start = jnp.concatenate(
        [jnp.zeros((1,), jnp.int32), jnp.cumsum(tt).astype(jnp.int32)])
    total = item_start[NEXP]
    g = jnp.arange(GRID_G, dtype=jnp.int32)
    eid = jnp.searchsorted(item_start[1:], g, side='right').astype(jnp.int32)
    eid = jnp.minimum(eid, NEXP - 1)
    tid = starts[eid] // TILE + (g - item_start[eid])
    valid = (g < total).astype(jnp.int32)
    last = jnp.maximum(total - 1, 0)
    tid = jnp.where(valid > 0, tid, tid[last]).astype(jnp.int32)
    eid = jnp.where(valid > 0, eid, eid[last]).astype(jnp.int32)
    first = jnp.concatenate(
        [jnp.ones((1,), jnp.int32), (tid[1:] != tid[:-1]).astype(jnp.int32)])
    first = first * valid
    return tid, eid, first, valid


def _ffn_body(tid_ref, eid_ref, first_ref, valid_ref, st_ref, en_ref,
              xs_ref, w1_ref, w3_ref, w2_ref, out_ref):
    g = pl.program_id(0)

    @pl.when(valid_ref[g] > 0)
    def _():
        e = eid_ref[g]
        row0 = tid_ref[g] * TILE
        rows = row0 + lax.broadcasted_iota(jnp.int32, (TILE, 1), 0)
        mask = ((rows >= st_ref[e]) & (rows < en_ref[e])).astype(jnp.float32)
        x = (xs_ref[...] * mask).astype(jnp.bfloat16)
        w1 = w1_ref[0].astype(jnp.bfloat16)
        w3 = w3_ref[0].astype(jnp.bfloat16)
        w2 = w2_ref[0].astype(jnp.bfloat16)
        h1 = lax.dot_general(x, w1, (((1,), (1,)), ((), ())),
                             preferred_element_type=jnp.float32)
        h3 = lax.dot_general(x, w3, (((1,), (1,)), ((), ())),
                             preferred_element_type=jnp.float32)
        hh = (h1 * jax.nn.sigmoid(h1) * h3).astype(jnp.bfloat16)
        contrib = lax.dot_general(hh, w2, (((1,), (1,)), ((), ())),
                                  preferred_element_type=jnp.float32)

        @pl.when(first_ref[g] > 0)
        def _():
            out_ref[...] = contrib

        @pl.when(first_ref[g] == 0)
        def _():
            out_ref[...] += contrib


def _ffn_call(tid, eid, first, valid, starts, ends, xs, exp_w1, exp_w3, exp_w2):
    grid_spec = pltpu.PrefetchScalarGridSpec(
        num_scalar_prefetch=6,
        grid=(GRID_G,),
        in_specs=[
            pl.BlockSpec((TILE, DIM), lambda g, tid, eid, f, v, st, en: (tid[g], 0)),
            pl.BlockSpec((1, HID, DIM), lambda g, tid, eid, f, v, st, en: (eid[g], 0, 0)),
            pl.BlockSpec((1, HID, DIM), lambda g, tid, eid, f, v, st, en: (eid[g], 0, 0)),
            pl.BlockSpec((1, DIM, HID), lambda g, tid, eid, f, v, st, en: (eid[g], 0, 0)),
        ],
        out_specs=pl.BlockSpec((TILE, DIM), lambda g, tid, eid, f, v, st, en: (tid[g], 0)),
    )
    return pl.pallas_call(
        _ffn_body,
        grid_spec=grid_spec,
        out_shape=jax.ShapeDtypeStruct((ROWS, DIM), jnp.float32),
        compiler_params=pltpu.CompilerParams(
            dimension_semantics=("arbitrary",)),
    )(tid, eid, first, valid, starts, ends, xs, exp_w1, exp_w3, exp_w2)


def _shared_body(x_ref, w1_ref, w2_ref, w3_ref, out_ref):
    x = x_ref[...]
    acc = None
    for s in range(NSHARED):
        h1 = lax.dot_general(x, w1_ref[s], (((1,), (1,)), ((), ())),
                             preferred_element_type=jnp.float32)
        h3 = lax.dot_general(x, w3_ref[s], (((1,), (1,)), ((), ())),
                             preferred_element_type=jnp.float32)
        hh = h1 * jax.nn.sigmoid(h1) * h3
        o = lax.dot_general(hh, w2_ref[s], (((1,), (1,)), ((), ())),
                            preferred_element_type=jnp.float32)
        acc = o if acc is None else acc + o
    out_ref[...] = acc * (1.0 / NSHARED)


def _shared_call(xf, shared_w1, shared_w2, shared_w3):
    nst = SEQ // SHTILE
    return pl.pallas_call(
        _shared_body,
        grid=(nst,),
        in_specs=[
            pl.BlockSpec((SHTILE, DIM), lambda i: (i, 0)),
            pl.BlockSpec((NSHARED, HID, DIM), lambda i: (0, 0, 0)),
            pl.BlockSpec((NSHARED, DIM, HID), lambda i: (0, 0, 0)),
            pl.BlockSpec((NSHARED, HID, DIM), lambda i: (0, 0, 0)),
        ],
        out_specs=pl.BlockSpec((SHTILE, DIM), lambda i: (i, 0)),
        out_shape=jax.ShapeDtypeStruct((SEQ, DIM), jnp.float32),
    )(xf, shared_w1, shared_w2, shared_w3)


def _dispatch_body(xf_hbm, dst0_hbm, dst1_hbm, xs_hbm, idx_v, rows_v, sem):
    wid = lax.axis_index("s") * NCORES + lax.axis_index("c")
    base = wid * TOKW
    pltpu.sync_copy(xf_hbm.at[pl.ds(base, TOKW)], rows_v)
    pltpu.sync_copy(dst0_hbm.at[wid], idx_v)
    pltpu.async_copy(rows_v, xs_hbm.at[idx_v], sem).wait()
    pltpu.sync_copy(dst1_hbm.at[wid], idx_v)
    pltpu.async_copy(rows_v, xs_hbm.at[idx_v], sem).wait()


def _dispatch_call(xf, dst0w, dst1w):
    mesh = plsc.VectorSubcoreMesh(core_axis_name="c", subcore_axis_name="s")
    f = pl.kernel(
        _dispatch_body,
        out_type=jax.ShapeDtypeStruct((ROWS, DIM), jnp.float32),
        mesh=mesh,
        scratch_types=[
            pltpu.VMEM((TOKW,), jnp.int32),
            pltpu.VMEM((TOKW, DIM), jnp.float32),
            pltpu.SemaphoreType.DMA,
        ],
    )
    return f(xf, dst0w, dst1w)


def _combine_body(ys_hbm, sh_hbm, dst0_hbm, dst1_hbm, w0x_hbm, w1x_hbm,
                  out_hbm, idx0_v, idx1_v, w0_v, w1_v, acc_v, tmp_v, sem):
    wid = lax.axis_index("s") * NCORES + lax.axis_index("c")
    base = wid * TOKW
    half = TOKW // 2
    pltpu.sync_copy(dst0_hbm.at[wid], idx0_v)
    pltpu.sync_copy(dst1_hbm.at[wid], idx1_v)
    pltpu.sync_copy(w0x_hbm.at[pl.ds(base, TOKW)], w0_v)
    pltpu.sync_copy(w1x_hbm.at[pl.ds(base, TOKW)], w1_v)

    for h in range(2):
        hb = h * half
        pltpu.sync_copy(sh_hbm.at[pl.ds(base + hb, half)], acc_v)

        for slot in range(2):
            idx_v = idx0_v if slot == 0 else idx1_v
            wv = w0_v if slot == 0 else w1_v
            pltpu.async_copy(ys_hbm.at[idx_v.at[h]], tmp_v, sem).wait()

            def row_body(r, _, wv=wv, hb=hb):
                wb = wv[hb + r]
                for c in range(DIM // 16):
                    sl = pl.ds(c * 16, 16)
                    acc_v[r, sl] = acc_v[r, sl] + wb * tmp_v[r, sl]
                return 0

            lax.fori_loop(0, half, row_body, 0)

        pltpu.sync_copy(acc_v, out_hbm.at[pl.ds(base + hb, half)])


def _combine_call(ys, shared, dst0h, dst1h, w0x, w1x):
    mesh = plsc.VectorSubcoreMesh(core_axis_name="c", subcore_axis_name="s")
    half = TOKW // 2
    f = pl.kernel(
        _combine_body,
        out_type=jax.ShapeDtypeStruct((SEQ, DIM), jnp.float32),
        mesh=mesh,
        scratch_types=[
            pltpu.VMEM((2, half), jnp.int32),
            pltpu.VMEM((2, half), jnp.int32),
            pltpu.VMEM((TOKW, 16), jnp.float32),
            pltpu.VMEM((TOKW, 16), jnp.float32),
            pltpu.VMEM((half, DIM), jnp.float32),
            pltpu.VMEM((half, DIM), jnp.float32),
            pltpu.SemaphoreType.DMA,
        ],
    )
    return f(ys, shared, dst0h, dst1h, w0x, w1x)


def kernel(x, gate_emb, gate_bias, shared_w1, shared_w2, shared_w3,
           exp_w1, exp_w2, exp_w3):
    b, s, d = x.shape
    xf = x.reshape(SEQ, DIM)

    w0x, w1x, dst0, dst1, cnt, off, ent = _route_call(
        xf, gate_emb, gate_bias.reshape(1, NEXP))

    shared = _shared_call(xf, shared_w1, shared_w2, shared_w3)

    dst0f = dst0[:, 0]
    dst1f = dst1[:, 0]
    xs = _dispatch_call(xf, dst0f.reshape(NW, TOKW), dst1f.reshape(NW, TOKW))

    counts = cnt[0]
    starts = off[0]
    tid, eid, first, valid = _work_items(counts, starts)
    ends = starts + counts
    ys = _ffn_call(tid, eid, first, valid, starts, ends,
                   xs, exp_w1, exp_w3, exp_w2)

    half = TOKW // 2
    out = _combine_call(ys, shared,
                        dst0f.reshape(NW, 2, half), dst1f.reshape(NW, 2, half),
                        w0x, w1x)

    aux_loss = jnp.asarray(0.0, dtype=x.dtype)
    return out.reshape(b, s, d), aux_loss, ent[0, 0]


# speedup vs baseline: 6.7005x; 1.0015x over previous
import functools

import jax
import jax.numpy as jnp
from jax import lax
from jax.experimental import pallas as pl
from jax.experimental.pallas import tpu as pltpu
from jax.experimental.pallas import tpu_sc as plsc

SEQ = 2048
DIM = 1024
NEXP = 64
HID = 256
NSHARED = 2
ROWS = 2 * SEQ
TILE = 128
NTILES = ROWS // TILE
GRID_G = NTILES + NEXP - 1
NCORES = 2
NSUB = 16
NW = NCORES * NSUB
TOKW = SEQ // NW
SHTILE = 256


def _route_body(x_ref, ge_ref, gb_ref, w0x_ref, w1x_ref, dst0_ref, dst1_ref,
                cnt_ref, off_ref, ent_ref):
    xf = x_ref[...]
    ge = ge_ref[...]
    logits = lax.dot_general(xf, ge, (((1,), (1,)), ((), ())),
                             preferred_element_type=jnp.float32)
    scores = jax.nn.sigmoid(logits) + gb_ref[...]

    eidx = lax.broadcasted_iota(jnp.int32, (SEQ, NEXP), 1)
    m1 = jnp.max(scores, axis=1, keepdims=True)
    i1 = jnp.min(jnp.where(scores == m1, eidx, NEXP), axis=1, keepdims=True)
    scores2 = jnp.where(eidx == i1, -jnp.inf, scores)
    m2 = jnp.max(scores2, axis=1, keepdims=True)
    i2 = jnp.min(jnp.where(scores2 == m2, eidx, NEXP), axis=1, keepdims=True)

    denom = m1 + m2
    w0 = m1 / denom
    w1 = m2 / denom
    ent = -(w0 * jnp.log(w0) + w1 * jnp.log(w1))
    ent_ref[...] = jnp.broadcast_to(jnp.mean(ent), (1, 1))

    oh0 = (eidx == i1).astype(jnp.float32)
    oh1 = (eidx == i2).astype(jnp.float32)
    comb = oh0 + oh1

    ri = lax.broadcasted_iota(jnp.int32, (SEQ, SEQ), 0)
    ci = lax.broadcasted_iota(jnp.int32, (SEQ, SEQ), 1)
    tri = (ri > ci).astype(jnp.bfloat16)
    cex = lax.dot_general(tri, comb.astype(jnp.bfloat16),
                          (((1,), (0,)), ((), ())),
                          preferred_element_type=jnp.float32)
    counts = cex[SEQ - 1:SEQ, :] + comb[SEQ - 1:SEQ, :]

    inc = counts
    for sh in (1, 2, 4, 8, 16, 32):
        shifted = jnp.concatenate(
            [jnp.zeros((1, sh), jnp.float32), inc[:, :NEXP - sh]], axis=1)
        inc = inc + shifted
    off = inc - counts

    pos = off + cex
    dst0 = jnp.sum(pos * oh0, axis=1, keepdims=True)
    dst1 = jnp.sum(pos * oh1, axis=1, keepdims=True)

    cnt_ref[...] = jnp.round(counts).astype(jnp.int32)
    off_ref[...] = jnp.round(off).astype(jnp.int32)
    dst0_ref[...] = jnp.round(dst0).astype(jnp.int32)
    dst1_ref[...] = jnp.round(dst1).astype(jnp.int32)
    w0x_ref[...] = jnp.broadcast_to(w0, (SEQ, 16))
    w1x_ref[...] = jnp.broadcast_to(w1, (SEQ, 16))


def _route_call(xf, gate_emb, gate_bias2d):
    out_shape = (
        jax.ShapeDtypeStruct((SEQ, 16), jnp.float32),
        jax.ShapeDtypeStruct((SEQ, 16), jnp.float32),
        jax.ShapeDtypeStruct((SEQ, 1), jnp.int32),
        jax.ShapeDtypeStruct((SEQ, 1), jnp.int32),
        jax.ShapeDtypeStruct((1, NEXP), jnp.int32),
        jax.ShapeDtypeStruct((1, NEXP), jnp.int32),
        jax.ShapeDtypeStruct((1, 1), jnp.float32),
    )
    return pl.pallas_call(_route_body, out_shape=out_shape)(
        xf, gate_emb, gate_bias2d)


def _work_items(counts, starts):
    ends = starts + counts
    tt = jnp.where(counts > 0, (ends - 1) // TILE - starts // TILE + 1, 0)
    item_start = jnp.concatenate(
        [jnp.zeros((1,), jnp.int32), jnp.cumsum(tt).astype(jnp.int32)])
    total = item_start[NEXP]
    g = jnp.arange(GRID_G, dtype=jnp.int32)
    eid = jnp.searchsorted(item_start[1:], g, side='right').astype(jnp.int32)
    eid = jnp.minimum(eid, NEXP - 1)
    tid = starts[eid] // TILE + (g - item_start[eid])
    valid = (g < total).astype(jnp.int32)
    last = jnp.maximum(total - 1, 0)
    tid = jnp.where(valid > 0, tid, tid[last]).astype(jnp.int32)
    eid = jnp.where(valid > 0, eid, eid[last]).astype(jnp.int32)
    first = jnp.concatenate(
        [jnp.ones((1,), jnp.int32), (tid[1:] != tid[:-1]).astype(jnp.int32)])
    first = first * valid
    return tid, eid, first, valid


def _ffn_body(tid_ref, eid_ref, first_ref, valid_ref, st_ref, en_ref,
              xs_ref, w1_ref, w3_ref, w2_ref, out_ref):
    g = pl.program_id(0)

    @pl.when(valid_ref[g] > 0)
    def _():
        e = eid_ref[g]
        row0 = tid_ref[g] * TILE
        rows = row0 + lax.broadcasted_iota(jnp.int32, (TILE, 1), 0)
        mask = ((rows >= st_ref[e]) & (rows < en_ref[e])).astype(jnp.float32)
        x = xs_ref[...] * mask
        h1 = lax.dot_general(x, w1_ref[0], (((1,), (1,)), ((), ())),
                             preferred_element_type=jnp.float32)
        h3 = lax.dot_general(x, w3_ref[0], (((1,), (1,)), ((), ())),
                             preferred_element_type=jnp.float32)
        hh = h1 * jax.nn.sigmoid(h1) * h3
        contrib = lax.dot_general(hh, w2_ref[0], (((1,), (1,)), ((), ())),
                                  preferred_element_type=jnp.float32)

        @pl.when(first_ref[g] > 0)
        def _():
            out_ref[...] = contrib

        @pl.when(first_ref[g] == 0)
        def _():
            out_ref[...] += contrib


def _ffn_call(tid, eid, first, valid, starts, ends, xs, exp_w1, exp_w3, exp_w2):
    grid_spec = pltpu.PrefetchScalarGridSpec(
        num_scalar_prefetch=6,
        grid=(GRID_G,),
        in_specs=[
            pl.BlockSpec((TILE, DIM), lambda g, tid, eid, f, v, st, en: (tid[g], 0)),
            pl.BlockSpec((1, HID, DIM), lambda g, tid, eid, f, v, st, en: (eid[g], 0, 0)),
            pl.BlockSpec((1, HID, DIM), lambda g, tid, eid, f, v, st, en: (eid[g], 0, 0)),
            pl.BlockSpec((1, DIM, HID), lambda g, tid, eid, f, v, st, en: (eid[g], 0, 0)),
        ],
        out_specs=pl.BlockSpec((TILE, DIM), lambda g, tid, eid, f, v, st, en: (tid[g], 0)),
    )
    return pl.pallas_call(
        _ffn_body,
        grid_spec=grid_spec,
        out_shape=jax.ShapeDtypeStruct((ROWS, DIM), jnp.float32),
        compiler_params=pltpu.CompilerParams(
            dimension_semantics=("arbitrary",)),
    )(tid, eid, first, valid, starts, ends, xs, exp_w1, exp_w3, exp_w2)


def _shared_body(x_ref, w1_ref, w2_ref, w3_ref, out_ref):
    x = x_ref[...]
    acc = None
    for s in range(NSHARED):
        h1 = lax.dot_general(x, w1_ref[s], (((1,), (1,)), ((), ())),
                             preferred_element_type=jnp.float32)
        h3 = lax.dot_general(x, w3_ref[s], (((1,), (1,)), ((), ())),
                             preferred_element_type=jnp.float32)
        hh = h1 * jax.nn.sigmoid(h1) * h3
        o = lax.dot_general(hh, w2_ref[s], (((1,), (1,)), ((), ())),
                            preferred_element_type=jnp.float32)
        acc = o if acc is None else acc + o
    out_ref[...] = acc * (1.0 / NSHARED)


def _shared_call(xf, shared_w1, shared_w2, shared_w3):
    nst = SEQ // SHTILE
    return pl.pallas_call(
        _shared_body,
        grid=(nst,),
        in_specs=[
            pl.BlockSpec((SHTILE, DIM), lambda i: (i, 0)),
            pl.BlockSpec((NSHARED, HID, DIM), lambda i: (0, 0, 0)),
            pl.BlockSpec((NSHARED, DIM, HID), lambda i: (0, 0, 0)),
            pl.BlockSpec((NSHARED, HID, DIM), lambda i: (0, 0, 0)),
        ],
        out_specs=pl.BlockSpec((SHTILE, DIM), lambda i: (i, 0)),
        out_shape=jax.ShapeDtypeStruct((SEQ, DIM), jnp.float32),
    )(xf, shared_w1, shared_w2, shared_w3)


def _dispatch_body(xf_hbm, dst0_hbm, dst1_hbm, xs_hbm, idx_v, rows_v, sem):
    wid = lax.axis_index("s") * NCORES + lax.axis_index("c")
    base = wid * TOKW
    pltpu.sync_copy(xf_hbm.at[pl.ds(base, TOKW)], rows_v)
    pltpu.sync_copy(dst0_hbm.at[wid], idx_v)
    pltpu.async_copy(rows_v, xs_hbm.at[idx_v], sem).wait()
    pltpu.sync_copy(dst1_hbm.at[wid], idx_v)
    pltpu.async_copy(rows_v, xs_hbm.at[idx_v], sem).wait()


def _dispatch_call(xf, dst0w, dst1w):
    mesh = plsc.VectorSubcoreMesh(core_axis_name="c", subcore_axis_name="s")
    f = pl.kernel(
        _dispatch_body,
        out_type=jax.ShapeDtypeStruct((ROWS, DIM), jnp.float32),
        mesh=mesh,
        scratch_types=[
            pltpu.VMEM((TOKW,), jnp.int32),
            pltpu.VMEM((TOKW, DIM), jnp.float32),
            pltpu.SemaphoreType.DMA,
        ],
    )
    return f(xf, dst0w, dst1w)


def _combine_body(ys_hbm, sh_hbm, dst0_hbm, dst1_hbm, w0x_hbm, w1x_hbm,
                  out_hbm, idx0_v, idx1_v, w0_v, w1_v, acc_v, tmp_v, sem):
    wid = lax.axis_index("s") * NCORES + lax.axis_index("c")
    base = wid * TOKW
    half = TOKW // 2
    pltpu.sync_copy(dst0_hbm.at[wid], idx0_v)
    pltpu.sync_copy(dst1_hbm.at[wid], idx1_v)
    pltpu.sync_copy(w0x_hbm.at[pl.ds(base, TOKW)], w0_v)
    pltpu.sync_copy(w1x_hbm.at[pl.ds(base, TOKW)], w1_v)

    for h in range(2):
        hb = h * half
        pltpu.sync_copy(sh_hbm.at[pl.ds(base + hb, half)], acc_v)

        for slot in range(2):
            idx_v = idx0_v if slot == 0 else idx1_v
            wv = w0_v if slot == 0 else w1_v
            pltpu.async_copy(ys_hbm.at[idx_v.at[h]], tmp_v, sem).wait()

            def row_body(r, _, wv=wv, hb=hb):
                wb = wv[hb + r]
                for c in range(DIM // 16):
                    sl = pl.ds(c * 16, 16)
                    acc_v[r, sl] = acc_v[r, sl] + wb * tmp_v[r, sl]
                return 0

            lax.fori_loop(0, half, row_body, 0)

        pltpu.sync_copy(acc_v, out_hbm.at[pl.ds(base + hb, half)])


def _combine_call(ys, shared, dst0h, dst1h, w0x, w1x):
    mesh = plsc.VectorSubcoreMesh(core_axis_name="c", subcore_axis_name="s")
    half = TOKW // 2
    f = pl.kernel(
        _combine_body,
        out_type=jax.ShapeDtypeStruct((SEQ, DIM), jnp.float32),
        mesh=mesh,
        scratch_types=[
            pltpu.VMEM((2, half), jnp.int32),
            pltpu.VMEM((2, half), jnp.int32),
            pltpu.VMEM((TOKW, 16), jnp.float32),
            pltpu.VMEM((TOKW, 16), jnp.float32),
            pltpu.VMEM((half, DIM), jnp.float32),
            pltpu.VMEM((half, DIM), jnp.float32),
            pltpu.SemaphoreType.DMA,
        ],
    )
    return f(ys, shared, dst0h, dst1h, w0x, w1x)


def kernel(x, gate_emb, gate_bias, shared_w1, shared_w2, shared_w3,
           exp_w1, exp_w2, exp_w3):
    b, s, d = x.shape
    xf = x.reshape(SEQ, DIM)

    w0x, w1x, dst0, dst1, cnt, off, ent = _route_call(
        xf, gate_emb, gate_bias.reshape(1, NEXP))

    shared = _shared_call(xf, shared_w1, shared_w2, shared_w3)

    dst0f = dst0[:, 0]
    dst1f = dst1[:, 0]
    xs = _dispatch_call(xf, dst0f.reshape(NW, TOKW), dst1f.reshape(NW, TOKW))

    counts = cnt[0]
    starts = off[0]
    tid, eid, first, valid = _work_items(counts, starts)
    ends = starts + counts
    ys = _ffn_call(tid, eid, first, valid, starts, ends,
                   xs, exp_w1, exp_w3, exp_w2)

    half = TOKW // 2
    out = _combine_call(ys, shared,
                        dst0f.reshape(NW, 2, half), dst1f.reshape(NW, 2, half),
                        w0x, w1x)

    aux_loss = jnp.asarray(0.0, dtype=x.dtype)
    return out.reshape(b, s, d), aux_loss, ent[0, 0]


# speedup vs baseline: 7.7464x; 1.1561x over previous
import functools

import jax
import jax.numpy as jnp
from jax import lax
from jax.experimental import pallas as pl
from jax.experimental.pallas import tpu as pltpu
from jax.experimental.pallas import tpu_sc as plsc

SEQ = 2048
DIM = 1024
NEXP = 64
HID = 256
NSHARED = 2
ROWS = 2 * SEQ
TILE = 128
GRID_G = 95
PADROWS = GRID_G * TILE
GPAD = 128
NCORES = 2
NSUB = 16
NW = NCORES * NSUB
TOKW = SEQ // NW
HALF = TOKW // 2
SHTILE = 256


def _excl_cumsum_lanes(v):
    inc = v
    for sh in (1, 2, 4, 8, 16, 32):
        inc = inc + jnp.concatenate(
            [jnp.zeros((1, sh), jnp.float32), inc[:, :NEXP - sh]], axis=1)
    return inc - v, inc


def _route_body(x_ref, ge_ref, gb_ref, w0x_ref, w1x_ref, dst0_ref, dst1_ref,
                eid_ref, val_ref, tot_ref, ent_ref):
    xf = x_ref[...]
    ge = ge_ref[...]
    logits = lax.dot_general(xf, ge, (((1,), (1,)), ((), ())),
                             preferred_element_type=jnp.float32)
    scores = jax.nn.sigmoid(logits) + gb_ref[...]

    eidx = lax.broadcasted_iota(jnp.int32, (SEQ, NEXP), 1)
    m1 = jnp.max(scores, axis=1, keepdims=True)
    i1 = jnp.min(jnp.where(scores == m1, eidx, NEXP), axis=1, keepdims=True)
    scores2 = jnp.where(eidx == i1, -jnp.inf, scores)
    m2 = jnp.max(scores2, axis=1, keepdims=True)
    i2 = jnp.min(jnp.where(scores2 == m2, eidx, NEXP), axis=1, keepdims=True)

    denom = m1 + m2
    w0 = m1 / denom
    w1 = m2 / denom
    ent = -(w0 * jnp.log(w0) + w1 * jnp.log(w1))
    ent_ref[...] = jnp.broadcast_to(jnp.mean(ent), (1, 1))

    oh0 = (eidx == i1).astype(jnp.float32)
    oh1 = (eidx == i2).astype(jnp.float32)
    comb = oh0 + oh1

    ri = lax.broadcasted_iota(jnp.int32, (SEQ, SEQ), 0)
    ci = lax.broadcasted_iota(jnp.int32, (SEQ, SEQ), 1)
    tri = (ri > ci).astype(jnp.bfloat16)
    cex = lax.dot_general(tri, comb.astype(jnp.bfloat16),
                          (((1,), (0,)), ((), ())),
                          preferred_element_type=jnp.float32)
    counts = cex[SEQ - 1:SEQ, :] + comb[SEQ - 1:SEQ, :]

    pc = jnp.ceil(counts * (1.0 / TILE)) * TILE
    astart, _ = _excl_cumsum_lanes(pc)
    tt = pc * (1.0 / TILE)
    it_excl, it_inc = _excl_cumsum_lanes(tt)
    total = it_inc[:, NEXP - 1:NEXP]

    pos = astart + cex
    dst0 = jnp.sum(pos * oh0, axis=1, keepdims=True)
    dst1 = jnp.sum(pos * oh1, axis=1, keepdims=True)

    gi = lax.broadcasted_iota(jnp.int32, (GPAD, 1), 0).astype(jnp.float32)
    eid = jnp.sum((it_inc <= gi).astype(jnp.float32), axis=1, keepdims=True)
    eid = jnp.minimum(eid, NEXP - 1)
    lastexp = jnp.sum((it_inc < total).astype(jnp.float32),
                      axis=1, keepdims=True)
    valid = (gi < total).astype(jnp.int32)
    eid = jnp.where(valid > 0, eid, lastexp)

    dst0_ref[...] = jnp.round(dst0).astype(jnp.int32)
    dst1_ref[...] = jnp.round(dst1).astype(jnp.int32)
    eid_ref[...] = jnp.round(eid).astype(jnp.int32)
    val_ref[...] = valid
    tot_ref[...] = jnp.round(total).astype(jnp.int32)
    w0x_ref[...] = jnp.broadcast_to(w0, (SEQ, 128))
    w1x_ref[...] = jnp.broadcast_to(w1, (SEQ, 128))


def _route_call(xf, gate_emb, gate_bias2d):
    out_shape = (
        jax.ShapeDtypeStruct((SEQ, 128), jnp.float32),
        jax.ShapeDtypeStruct((SEQ, 128), jnp.float32),
        jax.ShapeDtypeStruct((SEQ, 1), jnp.int32),
        jax.ShapeDtypeStruct((SEQ, 1), jnp.int32),
        jax.ShapeDtypeStruct((GPAD, 1), jnp.int32),
        jax.ShapeDtypeStruct((GPAD, 1), jnp.int32),
        jax.ShapeDtypeStruct((1, 1), jnp.int32),
        jax.ShapeDtypeStruct((1, 1), jnp.float32),
    )
    return pl.pallas_call(_route_body, out_shape=out_shape)(
        xf, gate_emb, gate_bias2d)


def _ffn_body(eid_ref, val_ref, tot_ref, xs_ref, ws_ref, w1_ref, w3_ref,
              w2_ref, out_ref):
    g = pl.program_id(0)

    @pl.when(val_ref[g, 0] > 0)
    def _():
        x = xs_ref[...]
        h1 = lax.dot_general(x, w1_ref[0], (((1,), (1,)), ((), ())),
                             preferred_element_type=jnp.float32)
        h3 = lax.dot_general(x, w3_ref[0], (((1,), (1,)), ((), ())),
                             preferred_element_type=jnp.float32)
        hh = h1 * jax.nn.sigmoid(h1) * h3
        contrib = lax.dot_general(hh, w2_ref[0], (((1,), (1,)), ((), ())),
                                  preferred_element_type=jnp.float32)
        out_ref[...] = contrib * ws_ref[:, :1]


def _tile_map(g, eid, val, tot):
    return (jnp.minimum(g, tot[0, 0] - 1), 0)


def _ffn_call(eidv, validv, tot, xs, ws, exp_w1, exp_w3, exp_w2):
    grid_spec = pltpu.PrefetchScalarGridSpec(
        num_scalar_prefetch=3,
        grid=(GRID_G,),
        in_specs=[
            pl.BlockSpec((TILE, DIM), _tile_map),
            pl.BlockSpec((TILE, 128), _tile_map),
            pl.BlockSpec((1, HID, DIM), lambda g, eid, val, tot: (eid[g, 0], 0, 0)),
            pl.BlockSpec((1, HID, DIM), lambda g, eid, val, tot: (eid[g, 0], 0, 0)),
            pl.BlockSpec((1, DIM, HID), lambda g, eid, val, tot: (eid[g, 0], 0, 0)),
        ],
        out_specs=pl.BlockSpec((TILE, DIM), _tile_map),
    )
    return pl.pallas_call(
        _ffn_body,
        grid_spec=grid_spec,
        out_shape=jax.ShapeDtypeStruct((PADROWS, DIM), jnp.float32),
        compiler_params=pltpu.CompilerParams(
            dimension_semantics=("arbitrary",)),
    )(eidv, validv, tot, xs, ws, exp_w1, exp_w3, exp_w2)


def _shared_body(x_ref, w1_ref, w2_ref, w3_ref, out_ref):
    x = x_ref[...]
    acc = None
    for s in range(NSHARED):
        h1 = lax.dot_general(x, w1_ref[s], (((1,), (1,)), ((), ())),
                             preferred_element_type=jnp.float32)
        h3 = lax.dot_general(x, w3_ref[s], (((1,), (1,)), ((), ())),
                             preferred_element_type=jnp.float32)
        hh = h1 * jax.nn.sigmoid(h1) * h3
        o = lax.dot_general(hh, w2_ref[s], (((1,), (1,)), ((), ())),
                            preferred_element_type=jnp.float32)
        acc = o if acc is None else acc + o
    out_ref[...] = acc * (1.0 / NSHARED)


def _shared_call(xf, shared_w1, shared_w2, shared_w3):
    nst = SEQ // SHTILE
    return pl.pallas_call(
        _shared_body,
        grid=(nst,),
        in_specs=[
            pl.BlockSpec((SHTILE, DIM), lambda i: (i, 0)),
            pl.BlockSpec((NSHARED, HID, DIM), lambda i: (0, 0, 0)),
            pl.BlockSpec((NSHARED, DIM, HID), lambda i: (0, 0, 0)),
            pl.BlockSpec((NSHARED, HID, DIM), lambda i: (0, 0, 0)),
        ],
        out_specs=pl.BlockSpec((SHTILE, DIM), lambda i: (i, 0)),
        out_shape=jax.ShapeDtypeStruct((SEQ, DIM), jnp.float32),
    )(xf, shared_w1, shared_w2, shared_w3)


def _dispatch_body(xf_hbm, w0x_hbm, w1x_hbm, dst0_hbm, dst1_hbm,
                   xs_hbm, ws_hbm, idx_v, rows_v, wrows_v, sem):
    wid = lax.axis_index("s") * NCORES + lax.axis_index("c")
    base = wid * TOKW
    pltpu.sync_copy(xf_hbm.at[pl.ds(base, TOKW)], rows_v)
    pltpu.sync_copy(w0x_hbm.at[pl.ds(base, TOKW)], wrows_v)
    pltpu.sync_copy(dst0_hbm.at[wid], idx_v)
    pltpu.async_copy(rows_v, xs_hbm.at[idx_v], sem).wait()
    pltpu.async_copy(wrows_v, ws_hbm.at[idx_v], sem).wait()
    pltpu.sync_copy(w1x_hbm.at[pl.ds(base, TOKW)], wrows_v)
    pltpu.sync_copy(dst1_hbm.at[wid], idx_v)
    pltpu.async_copy(rows_v, xs_hbm.at[idx_v], sem).wait()
    pltpu.async_copy(wrows_v, ws_hbm.at[idx_v], sem).wait()


def _dispatch_call(xf, w0x, w1x, dst0w, dst1w):
    mesh = plsc.VectorSubcoreMesh(core_axis_name="c", subcore_axis_name="s")
    f = pl.kernel(
        _dispatch_body,
        out_type=(
            jax.ShapeDtypeStruct((PADROWS, DIM), jnp.float32),
            jax.ShapeDtypeStruct((PADROWS, 128), jnp.float32),
        ),
        mesh=mesh,
        scratch_types=[
            pltpu.VMEM((TOKW,), jnp.int32),
            pltpu.VMEM((TOKW, DIM), jnp.float32),
            pltpu.VMEM((TOKW, 128), jnp.float32),
            pltpu.SemaphoreType.DMA,
        ],
    )
    return f(xf, w0x, w1x, dst0w, dst1w)


def _combine_body(ys_hbm, sh_hbm, dstc_hbm, out_hbm, idx_v, acc_v, tmp_v, sem):
    wid = lax.axis_index("s") * NCORES + lax.axis_index("c")
    base = wid * TOKW
    pltpu.sync_copy(dstc_hbm.at[wid], idx_v)

    for h in range(2):
        hb = h * HALF
        pltpu.sync_copy(sh_hbm.at[pl.ds(base + hb, HALF)], acc_v)
        pltpu.async_copy(ys_hbm.at[idx_v.at[h]], tmp_v, sem).wait()

        def row_body(r, _):
            for c in range(DIM // 16):
                sl = pl.ds(c * 16, 16)
                acc_v[r, sl] = acc_v[r, sl] + tmp_v[r, sl] + tmp_v[HALF + r, sl]
            return 0

        lax.fori_loop(0, HALF, row_body, 0)
        pltpu.sync_copy(acc_v, out_hbm.at[pl.ds(base + hb, HALF)])


def _combine_call(ys, shared, dstc):
    mesh = plsc.VectorSubcoreMesh(core_axis_name="c", subcore_axis_name="s")
    f = pl.kernel(
        _combine_body,
        out_type=jax.ShapeDtypeStruct((SEQ, DIM), jnp.float32),
        mesh=mesh,
        scratch_types=[
            pltpu.VMEM((2, 2 * HALF), jnp.int32),
            pltpu.VMEM((HALF, DIM), jnp.float32),
            pltpu.VMEM((2 * HALF, DIM), jnp.float32),
            pltpu.SemaphoreType.DMA,
        ],
    )
    return f(ys, shared, dstc)


def kernel(x, gate_emb, gate_bias, shared_w1, shared_w2, shared_w3,
           exp_w1, exp_w2, exp_w3):
    b, s, d = x.shape
    xf = x.reshape(SEQ, DIM)

    (w0x, w1x, dst0, dst1, eidv, validv, tot, ent) = _route_call(
        xf, gate_emb, gate_bias.reshape(1, NEXP))

    shared = _shared_call(xf, shared_w1, shared_w2, shared_w3)

    dst0f = dst0[:, 0]
    dst1f = dst1[:, 0]
    xs, ws = _dispatch_call(xf, w0x, w1x,
                            dst0f.reshape(NW, TOKW), dst1f.reshape(NW, TOKW))

    ys = _ffn_call(eidv, validv, tot, xs, ws, exp_w1, exp_w3, exp_w2)

    dstc = jnp.concatenate(
        [dst0f.reshape(NW, 2, HALF), dst1f.reshape(NW, 2, HALF)], axis=2)
    out = _combine_call(ys, shared, dstc)

    aux_loss = jnp.asarray(0.0, dtype=x.dtype)
    return out.reshape(b, s, d), aux_loss, ent[0, 0]


# speedup vs baseline: 8.2264x; 1.0620x over previous
import functools

import jax
import jax.numpy as jnp
from jax import lax
from jax.experimental import pallas as pl
from jax.experimental.pallas import tpu as pltpu
from jax.experimental.pallas import tpu_sc as plsc

SEQ = 2048
DIM = 1024
NEXP = 64
HID = 256
NSHARED = 2
ROWS = 2 * SEQ
TILE = 256
GRID_G = ROWS // TILE + NEXP - 1
PADROWS = GRID_G * TILE
GPAD = 128
NCORES = 2
NSUB = 16
NW = NCORES * NSUB
TOKW = SEQ // NW
HALF = TOKW // 2
SHTILE = 256


def _excl_cumsum_lanes(v):
    inc = v
    for sh in (1, 2, 4, 8, 16, 32):
        inc = inc + jnp.concatenate(
            [jnp.zeros((1, sh), jnp.float32), inc[:, :NEXP - sh]], axis=1)
    return inc - v, inc


def _route_body(x_ref, ge_ref, gb_ref, w0x_ref, w1x_ref, dst0_ref, dst1_ref,
                eid_ref, val_ref, tot_ref, ent_ref):
    xf = x_ref[...]
    ge = ge_ref[...]
    logits = lax.dot_general(xf, ge, (((1,), (1,)), ((), ())),
                             preferred_element_type=jnp.float32)
    scores = jax.nn.sigmoid(logits) + gb_ref[...]

    eidx = lax.broadcasted_iota(jnp.int32, (SEQ, NEXP), 1)
    m1 = jnp.max(scores, axis=1, keepdims=True)
    i1 = jnp.min(jnp.where(scores == m1, eidx, NEXP), axis=1, keepdims=True)
    scores2 = jnp.where(eidx == i1, -jnp.inf, scores)
    m2 = jnp.max(scores2, axis=1, keepdims=True)
    i2 = jnp.min(jnp.where(scores2 == m2, eidx, NEXP), axis=1, keepdims=True)

    denom = m1 + m2
    w0 = m1 / denom
    w1 = m2 / denom
    ent = -(w0 * jnp.log(w0) + w1 * jnp.log(w1))
    ent_ref[...] = jnp.broadcast_to(jnp.mean(ent), (1, 1))

    oh0 = (eidx == i1).astype(jnp.float32)
    oh1 = (eidx == i2).astype(jnp.float32)
    comb = oh0 + oh1

    ri = lax.broadcasted_iota(jnp.int32, (SEQ, SEQ), 0)
    ci = lax.broadcasted_iota(jnp.int32, (SEQ, SEQ), 1)
    tri = (ri > ci).astype(jnp.bfloat16)
    cex = lax.dot_general(tri, comb.astype(jnp.bfloat16),
                          (((1,), (0,)), ((), ())),
                          preferred_element_type=jnp.float32)
    counts = cex[SEQ - 1:SEQ, :] + comb[SEQ - 1:SEQ, :]

    pc = jnp.ceil(counts * (1.0 / TILE)) * TILE
    astart, _ = _excl_cumsum_lanes(pc)
    tt = pc * (1.0 / TILE)
    it_excl, it_inc = _excl_cumsum_lanes(tt)
    total = it_inc[:, NEXP - 1:NEXP]

    pos = astart + cex
    dst0 = jnp.sum(pos * oh0, axis=1, keepdims=True)
    dst1 = jnp.sum(pos * oh1, axis=1, keepdims=True)

    gi = lax.broadcasted_iota(jnp.int32, (GPAD, 1), 0).astype(jnp.float32)
    eid = jnp.sum((it_inc <= gi).astype(jnp.float32), axis=1, keepdims=True)
    eid = jnp.minimum(eid, NEXP - 1)
    lastexp = jnp.sum((it_inc < total).astype(jnp.float32),
                      axis=1, keepdims=True)
    valid = (gi < total).astype(jnp.int32)
    eid = jnp.where(valid > 0, eid, lastexp)

    dst0_ref[...] = jnp.round(dst0).astype(jnp.int32)
    dst1_ref[...] = jnp.round(dst1).astype(jnp.int32)
    eid_ref[...] = jnp.round(eid).astype(jnp.int32)
    val_ref[...] = valid
    tot_ref[...] = jnp.round(total).astype(jnp.int32)
    w0x_ref[...] = jnp.broadcast_to(w0, (SEQ, 128))
    w1x_ref[...] = jnp.broadcast_to(w1, (SEQ, 128))


def _route_call(xf, gate_emb, gate_bias2d):
    out_shape = (
        jax.ShapeDtypeStruct((SEQ, 128), jnp.float32),
        jax.ShapeDtypeStruct((SEQ, 128), jnp.float32),
        jax.ShapeDtypeStruct((SEQ, 1), jnp.int32),
        jax.ShapeDtypeStruct((SEQ, 1), jnp.int32),
        jax.ShapeDtypeStruct((GPAD, 1), jnp.int32),
        jax.ShapeDtypeStruct((GPAD, 1), jnp.int32),
        jax.ShapeDtypeStruct((1, 1), jnp.int32),
        jax.ShapeDtypeStruct((1, 1), jnp.float32),
    )
    return pl.pallas_call(_route_body, out_shape=out_shape)(
        xf, gate_emb, gate_bias2d)


def _ffn_body(eid_ref, val_ref, tot_ref, xs_ref, ws_ref, w1_ref, w3_ref,
              w2_ref, out_ref):
    g = pl.program_id(0)

    @pl.when(val_ref[g, 0] > 0)
    def _():
        x = xs_ref[...]
        h1 = lax.dot_general(x, w1_ref[0], (((1,), (1,)), ((), ())),
                             preferred_element_type=jnp.float32)
        h3 = lax.dot_general(x, w3_ref[0], (((1,), (1,)), ((), ())),
                             preferred_element_type=jnp.float32)
        hh = h1 * jax.nn.sigmoid(h1) * h3
        contrib = lax.dot_general(hh, w2_ref[0], (((1,), (1,)), ((), ())),
                                  preferred_element_type=jnp.float32)
        out_ref[...] = contrib * ws_ref[:, :1]


def _tile_map(g, eid, val, tot):
    return (jnp.minimum(g, tot[0, 0] - 1), 0)


def _ffn_call(eidv, validv, tot, xs, ws, exp_w1, exp_w3, exp_w2):
    grid_spec = pltpu.PrefetchScalarGridSpec(
        num_scalar_prefetch=3,
        grid=(GRID_G,),
        in_specs=[
            pl.BlockSpec((TILE, DIM), _tile_map),
            pl.BlockSpec((TILE, 128), _tile_map),
            pl.BlockSpec((1, HID, DIM), lambda g, eid, val, tot: (eid[g, 0], 0, 0)),
            pl.BlockSpec((1, HID, DIM), lambda g, eid, val, tot: (eid[g, 0], 0, 0)),
            pl.BlockSpec((1, DIM, HID), lambda g, eid, val, tot: (eid[g, 0], 0, 0)),
        ],
        out_specs=pl.BlockSpec((TILE, DIM), _tile_map),
    )
    return pl.pallas_call(
        _ffn_body,
        grid_spec=grid_spec,
        out_shape=jax.ShapeDtypeStruct((PADROWS, DIM), jnp.float32),
        compiler_params=pltpu.CompilerParams(
            dimension_semantics=("arbitrary",)),
    )(eidv, validv, tot, xs, ws, exp_w1, exp_w3, exp_w2)


def _shared_body(x_ref, w1_ref, w2_ref, w3_ref, out_ref):
    x = x_ref[...]
    acc = None
    for s in range(NSHARED):
        h1 = lax.dot_general(x, w1_ref[s], (((1,), (1,)), ((), ())),
                             preferred_element_type=jnp.float32)
        h3 = lax.dot_general(x, w3_ref[s], (((1,), (1,)), ((), ())),
                             preferred_element_type=jnp.float32)
        hh = h1 * jax.nn.sigmoid(h1) * h3
        o = lax.dot_general(hh, w2_ref[s], (((1,), (1,)), ((), ())),
                            preferred_element_type=jnp.float32)
        acc = o if acc is None else acc + o
    out_ref[...] = acc * (1.0 / NSHARED)


def _shared_call(xf, shared_w1, shared_w2, shared_w3):
    nst = SEQ // SHTILE
    return pl.pallas_call(
        _shared_body,
        grid=(nst,),
        in_specs=[
            pl.BlockSpec((SHTILE, DIM), lambda i: (i, 0)),
            pl.BlockSpec((NSHARED, HID, DIM), lambda i: (0, 0, 0)),
            pl.BlockSpec((NSHARED, DIM, HID), lambda i: (0, 0, 0)),
            pl.BlockSpec((NSHARED, HID, DIM), lambda i: (0, 0, 0)),
        ],
        out_specs=pl.BlockSpec((SHTILE, DIM), lambda i: (i, 0)),
        out_shape=jax.ShapeDtypeStruct((SEQ, DIM), jnp.float32),
    )(xf, shared_w1, shared_w2, shared_w3)


def _dispatch_body(xf_hbm, w0x_hbm, w1x_hbm, dst0_hbm, dst1_hbm,
                   xs_hbm, ws_hbm, idx_v, rows_v, wrows_v, sem):
    wid = lax.axis_index("s") * NCORES + lax.axis_index("c")
    base = wid * TOKW
    pltpu.sync_copy(xf_hbm.at[pl.ds(base, TOKW)], rows_v)
    pltpu.sync_copy(w0x_hbm.at[pl.ds(base, TOKW)], wrows_v)
    pltpu.sync_copy(dst0_hbm.at[wid], idx_v)
    pltpu.async_copy(rows_v, xs_hbm.at[idx_v], sem).wait()
    pltpu.async_copy(wrows_v, ws_hbm.at[idx_v], sem).wait()
    pltpu.sync_copy(w1x_hbm.at[pl.ds(base, TOKW)], wrows_v)
    pltpu.sync_copy(dst1_hbm.at[wid], idx_v)
    pltpu.async_copy(rows_v, xs_hbm.at[idx_v], sem).wait()
    pltpu.async_copy(wrows_v, ws_hbm.at[idx_v], sem).wait()


def _dispatch_call(xf, w0x, w1x, dst0w, dst1w):
    mesh = plsc.VectorSubcoreMesh(core_axis_name="c", subcore_axis_name="s")
    f = pl.kernel(
        _dispatch_body,
        out_type=(
            jax.ShapeDtypeStruct((PADROWS, DIM), jnp.float32),
            jax.ShapeDtypeStruct((PADROWS, 128), jnp.float32),
        ),
        mesh=mesh,
        scratch_types=[
            pltpu.VMEM((TOKW,), jnp.int32),
            pltpu.VMEM((TOKW, DIM), jnp.float32),
            pltpu.VMEM((TOKW, 128), jnp.float32),
            pltpu.SemaphoreType.DMA,
        ],
    )
    return f(xf, w0x, w1x, dst0w, dst1w)


def _combine_body(ys_hbm, sh_hbm, dstc_hbm, out_hbm, idx_v, acc_v, tmp_v, sem):
    wid = lax.axis_index("s") * NCORES + lax.axis_index("c")
    base = wid * TOKW
    pltpu.sync_copy(dstc_hbm.at[wid], idx_v)

    for h in range(2):
        hb = h * HALF
        pltpu.sync_copy(sh_hbm.at[pl.ds(base + hb, HALF)], acc_v)
        pltpu.async_copy(ys_hbm.at[idx_v.at[h]], tmp_v, sem).wait()

        def row_body(r, _):
            for c in range(DIM // 16):
                sl = pl.ds(c * 16, 16)
                acc_v[r, sl] = acc_v[r, sl] + tmp_v[r, sl] + tmp_v[HALF + r, sl]
            return 0

        lax.fori_loop(0, HALF, row_body, 0)
        pltpu.sync_copy(acc_v, out_hbm.at[pl.ds(base + hb, HALF)])


def _combine_call(ys, shared, dstc):
    mesh = plsc.VectorSubcoreMesh(core_axis_name="c", subcore_axis_name="s")
    f = pl.kernel(
        _combine_body,
        out_type=jax.ShapeDtypeStruct((SEQ, DIM), jnp.float32),
        mesh=mesh,
        scratch_types=[
            pltpu.VMEM((2, 2 * HALF), jnp.int32),
            pltpu.VMEM((HALF, DIM), jnp.float32),
            pltpu.VMEM((2 * HALF, DIM), jnp.float32),
            pltpu.SemaphoreType.DMA,
        ],
    )
    return f(ys, shared, dstc)


def kernel(x, gate_emb, gate_bias, shared_w1, shared_w2, shared_w3,
           exp_w1, exp_w2, exp_w3):
    b, s, d = x.shape
    xf = x.reshape(SEQ, DIM)

    (w0x, w1x, dst0, dst1, eidv, validv, tot, ent) = _route_call(
        xf, gate_emb, gate_bias.reshape(1, NEXP))

    shared = _shared_call(xf, shared_w1, shared_w2, shared_w3)

    dst0f = dst0[:, 0]
    dst1f = dst1[:, 0]
    xs, ws = _dispatch_call(xf, w0x, w1x,
                            dst0f.reshape(NW, TOKW), dst1f.reshape(NW, TOKW))

    ys = _ffn_call(eidv, validv, tot, xs, ws, exp_w1, exp_w3, exp_w2)

    dstc = jnp.concatenate(
        [dst0f.reshape(NW, 2, HALF), dst1f.reshape(NW, 2, HALF)], axis=2)
    out = _combine_call(ys, shared, dstc)

    aux_loss = jnp.asarray(0.0, dtype=x.dtype)
    return out.reshape(b, s, d), aux_loss, ent[0, 0]
